# baseline (speedup 1.0000x reference)
import functools

import jax
import jax.numpy as jnp
from jax import lax
from jax.experimental import pallas as pl
from jax.experimental.pallas import tpu as pltpu
from jax.experimental.pallas import tpu_sc as plsc

_B, _V, _C, _H, _W = 4, 5, 128, 96, 96
_P = 10
_E = 1000
_HID = 128

_NC, _NS, _L = 2, 16, 16
_NW = _NC * _NS
_NODES = _B * _V * _P
_NPAD = 256
_NPT = _NPAD // _NW
_HW = _H * _W
_CHW = _C * _HW


def _sc_grid_sample(fm_flat, xs, ys):
    mesh = plsc.VectorSubcoreMesh(
        core_axis_name="c", subcore_axis_name="s",
        num_cores=_NC, num_subcores=_NS)

    @functools.partial(
        pl.kernel,
        out_type=jax.ShapeDtypeStruct((_NPAD, _C), jnp.float32),
        mesh=mesh,
        scratch_types=[
            pltpu.VMEM((_L,), jnp.float32),
            pltpu.VMEM((_L,), jnp.float32),
            pltpu.VMEM((_C,), jnp.int32),
            pltpu.VMEM((4, _C), jnp.int32),
            pltpu.VMEM((4, _C), jnp.float32),
            pltpu.VMEM((_NPT, _C), jnp.float32),
            pltpu.SemaphoreType.DMA,
        ],
    )
    def body(fm_hbm, xs_hbm, ys_hbm, out_hbm,
             xs_v, ys_v, c_off, idx_v, vals_v, out_v, sem):
        wid = lax.axis_index("s") * _NC + lax.axis_index("c")
        base_node = wid * _NPT

        pltpu.sync_copy(xs_hbm.at[pl.ds(base_node, _NPT)], xs_v.at[pl.ds(0, _NPT)])
        pltpu.sync_copy(ys_hbm.at[pl.ds(base_node, _NPT)], ys_v.at[pl.ds(0, _NPT)])
        xv = xs_v[...]
        yv = ys_v[...]

        for j in range(_C // _L):
            c_off[pl.ds(j * _L, _L)] = (
                lax.iota(jnp.int32, _L) + (j * _L)) * _HW

        for i in range(_NPT):
            node = base_node + i
            node_c = jnp.minimum(node, _NODES - 1)
            b = node_c // (_V * _P)
            v = (node_c % (_V * _P)) // _P
            base_map = (b * _V + v) * _CHW

            x = jnp.minimum(jnp.maximum(xv[i], 0.0), float(_W - 1))
            y = jnp.minimum(jnp.maximum(yv[i], 0.0), float(_H - 1))
            xr = x.astype(jnp.int32)
            yr = y.astype(jnp.int32)
            xr = xr - jnp.where(xr.astype(jnp.float32) > x, 1, 0)
            yr = yr - jnp.where(yr.astype(jnp.float32) > y, 1, 0)
            xi = jnp.minimum(xr, _W - 2)
            yi = jnp.minimum(yr, _H - 2)
            wx = x - xi.astype(jnp.float32)
            wy = y - yi.astype(jnp.float32)

            base00 = base_map + yi * _W + xi
            bases = (base00, base00 + 1, base00 + _W, base00 + _W + 1)
            for k in range(4):
                for j in range(_C // _L):
                    sl = pl.ds(j * _L, _L)
                    idx_v[k, sl] = c_off[sl] + bases[k]

            cps = [pltpu.async_copy(fm_hbm.at[idx_v.at[k]], vals_v.at[k], sem)
                   for k in range(4)]
            for cp in cps:
                cp.wait()

            w00 = (1.0 - wx) * (1.0 - wy)
            w01 = wx * (1.0 - wy)
            w10 = (1.0 - wx) * wy
            w11 = wx * wy
            for j in range(_C // _L):
                sl = pl.ds(j * _L, _L)
                out_v[i, sl] = (vals_v[0, sl] * w00 + vals_v[1, sl] * w01
                                + vals_v[2, sl] * w10 + vals_v[3, sl] * w11)

        pltpu.sync_copy(out_v, out_hbm.at[pl.ds(base_node, _NPT)])

    return body(fm_flat, xs, ys)


def _tc_gcn(nf, src, dst, es, el, ev, W1, W2, W3):
    ne = src.shape[0]

    def body(nf_ref, src_ref, dst_ref, es_ref, el_ref, ev_ref,
             w1_ref, w2_ref, w3_ref, out_ref):
        eps = 1e-12
        nfv = nf_ref[...]
        h = jnp.maximum(jnp.dot(nfv, w1_ref[...],
                                preferred_element_type=jnp.float32, precision=lax.Precision.HIGHEST), 0.0)
        es_v = es_ref[...]
        el_v = el_ref[...]
        mask = (ev_ref[...] > 0.0).astype(jnp.float32)

        n_iota = lax.broadcasted_iota(jnp.int32, (ne, _NPAD), 1)
        oh_src = (src_ref[...] == n_iota).astype(jnp.float32)
        oh_dst = (dst_ref[...] == n_iota).astype(jnp.float32)

        h_src = jnp.dot(oh_src, h, preferred_element_type=jnp.float32, precision=lax.Precision.HIGHEST)
        msg = h_src * (es_v * mask)
        agg = lax.dot_general(oh_dst, msg, (((0,), (0,)), ((), ())),
                              preferred_element_type=jnp.float32,
                              precision=lax.Precision.HIGHEST)
        h2 = jnp.maximum(jnp.dot(h + agg, w2_ref[...],
                                 preferred_element_type=jnp.float32, precision=lax.Precision.HIGHEST), 0.0)
        h2s = jnp.dot(oh_src, h2, preferred_element_type=jnp.float32, precision=lax.Precision.HIGHEST)
        h2d = jnp.dot(oh_dst, h2, preferred_element_type=jnp.float32, precision=lax.Precision.HIGHEST)

        w3v = w3_ref[...]
        logit = (jnp.dot(h2s * h2d, w3v[:_HID, :],
                         preferred_element_type=jnp.float32, precision=lax.Precision.HIGHEST)
                 + es_v * w3v[_HID, 0])
        preds = 1.0 / (1.0 + jnp.exp(-logit))
        loss = -(el_v * jnp.log(preds + eps)
                 + (1.0 - el_v) * jnp.log(1.0 - preds + eps))
        num_pos = jnp.sum(el_v * mask)
        num_neg = jnp.sum(mask) - num_pos
        mw = jnp.where(el_v > 0.0, 1.0 / (num_pos + eps),
                       1.0 / (num_neg + eps))
        out_ref[...] = jnp.sum(loss * mw * mask).reshape(1, 1)

    return pl.pallas_call(
        body,
        out_shape=jax.ShapeDtypeStruct((1, 1), jnp.float32),
    )(nf, src, dst, es, el, ev, W1, W2, W3)


def kernel(feature_maps, multiview_centers, edge_scores, edge_labels,
           edge_valid, edge_indices, W1, W2, W3):
    num_nodes = _V * _P

    fm_flat = feature_maps.reshape(-1)
    pts = multiview_centers[..., :2].reshape(-1, 2)
    xs = jnp.pad(pts[:, 0], (0, _NPAD - _NODES))
    ys = jnp.pad(pts[:, 1], (0, _NPAD - _NODES))

    offs = (jnp.arange(_B, dtype=edge_indices.dtype) * num_nodes)[:, None, None]
    ei = (edge_indices + offs).reshape(-1, 2).astype(jnp.int32)
    src = ei[:, 0:1]
    dst = ei[:, 1:2]
    es = edge_scores.reshape(-1, 1)
    el = edge_labels.reshape(-1, 1)
    ev = edge_valid.reshape(-1, 1)

    nf = _sc_grid_sample(fm_flat, xs, ys)

    out = _tc_gcn(nf, src, dst, es, el, ev, W1, W2, W3)
    return out[0, 0]


# speedup vs baseline: 1.4679x; 1.4679x over previous
import functools

import jax
import jax.numpy as jnp
from jax import lax
from jax.experimental import pallas as pl
from jax.experimental.pallas import tpu as pltpu
from jax.experimental.pallas import tpu_sc as plsc

_B, _V, _C, _H, _W = 4, 5, 128, 96, 96
_P = 10
_E = 1000
_HID = 128

_NC, _NS, _L = 2, 16, 16
_NW = _NC * _NS
_NODES = _B * _V * _P
_NPAD = 256
_NPT = _NPAD // _NW


def _sc_gather_rows(fm4, xs, ys):
    mesh = plsc.VectorSubcoreMesh(
        core_axis_name="c", subcore_axis_name="s",
        num_cores=_NC, num_subcores=_NS)

    @functools.partial(
        pl.kernel,
        out_type=(jax.ShapeDtypeStruct((2 * _NPAD, _C, _W), jnp.float32),
                  jax.ShapeDtypeStruct((2 * _NPAD, 128), jnp.float32)),
        mesh=mesh,
        scratch_types=[
            pltpu.VMEM((_L,), jnp.float32),
            pltpu.VMEM((_L,), jnp.float32),
            pltpu.VMEM((_C, _W), jnp.float32),
            pltpu.VMEM((_C, _W), jnp.float32),
            pltpu.VMEM((2, 128), jnp.float32),
            pltpu.SemaphoreType.DMA,
        ],
        compiler_params=pltpu.CompilerParams(use_tc_tiling_on_sc=True),
    )
    def body(fm4_hbm, xs_hbm, ys_hbm, rows_hbm, w_hbm,
             xs_v, ys_v, buf0_v, buf1_v, wv_v, sem):
        wid = lax.axis_index("s") * _NC + lax.axis_index("c")
        base_node = wid * _NPT
        pltpu.sync_copy(xs_hbm.at[pl.ds(base_node, _NPT)],
                        xs_v.at[pl.ds(0, _NPT)])
        pltpu.sync_copy(ys_hbm.at[pl.ds(base_node, _NPT)],
                        ys_v.at[pl.ds(0, _NPT)])
        xv = xs_v[...]
        yv = ys_v[...]
        for i in range(_NPT):
            node = base_node + i
            node_c = jnp.minimum(node, _NODES - 1)
            b = node_c // (_V * _P)
            v = (node_c % (_V * _P)) // _P
            bv = b * _V + v
            x = jnp.minimum(jnp.maximum(xv[i], 0.0), float(_W - 1))
            y = jnp.minimum(jnp.maximum(yv[i], 0.0), float(_H - 1))
            xr = x.astype(jnp.int32)
            yr = y.astype(jnp.int32)
            xr = xr - jnp.where(xr.astype(jnp.float32) > x, 1, 0)
            yr = yr - jnp.where(yr.astype(jnp.float32) > y, 1, 0)
            xi = jnp.minimum(xr, _W - 2)
            yi = jnp.minimum(yr, _H - 2)
            wx = x - xi.astype(jnp.float32)
            wy = y - yi.astype(jnp.float32)
            cp0 = pltpu.async_copy(fm4_hbm.at[bv, :, yi, :], buf0_v, sem)
            cp1 = pltpu.async_copy(fm4_hbm.at[bv, :, yi + 1, :], buf1_v, sem)
            for k in range(2):
                wyk = (1.0 - wy) if k == 0 else wy
                for j in range(128 // _L):
                    xloc = lax.iota(jnp.int32, _L) + (j * _L)
                    wvj = (jnp.where(xloc == xi, 1.0 - wx, 0.0)
                           + jnp.where(xloc == xi + 1, wx, 0.0)) * wyk
                    wv_v[k, pl.ds(j * _L, _L)] = wvj
            pltpu.sync_copy(wv_v, w_hbm.at[pl.ds(node * 2, 2)])
            cp0.wait()
            cp1.wait()
            pltpu.sync_copy(buf0_v, rows_hbm.at[node * 2])
            pltpu.sync_copy(buf1_v, rows_hbm.at[node * 2 + 1])

    return body(fm4, xs, ys)


def _tc_finish(rows, wvec, src, dst, es, el, ev, W1, W2, W3):
    ne = src.shape[0]
    hp = lax.Precision.HIGHEST

    ck = 32

    def extract_body(rows_ref, wv_ref, out_ref):
        out_ref[...] = jnp.sum(
            rows_ref[...] * wv_ref[...][:, None, :_W], axis=-1)

    contrib = pl.pallas_call(
        extract_body,
        grid=(2 * _NPAD // ck,),
        in_specs=[pl.BlockSpec((ck, _C, _W), lambda i: (i, 0, 0)),
                  pl.BlockSpec((ck, 128), lambda i: (i, 0))],
        out_specs=pl.BlockSpec((ck, _C), lambda i: (i, 0)),
        out_shape=jax.ShapeDtypeStruct((2 * _NPAD, _C), jnp.float32),
    )(rows, wvec)

    def body(cb_ref, src_ref, dst_ref, es_ref, el_ref, ev_ref,
             w1_ref, w2_ref, w3_ref, out_ref):
        eps = 1e-12
        nfv = cb_ref[...].reshape(_NPAD, 2, _C).sum(axis=1)

        h = jnp.maximum(jnp.dot(nfv, w1_ref[...],
                                preferred_element_type=jnp.float32,
                                precision=hp), 0.0)
        es_v = es_ref[...]
        el_v = el_ref[...]
        mask = (ev_ref[...] > 0.0).astype(jnp.float32)

        n_iota = lax.broadcasted_iota(jnp.int32, (ne, _NPAD), 1)
        oh_src = (src_ref[...] == n_iota).astype(jnp.float32)
        oh_dst = (dst_ref[...] == n_iota).astype(jnp.float32)

        h_src = jnp.dot(oh_src, h, preferred_element_type=jnp.float32,
                        precision=hp)
        msg = h_src * (es_v * mask)
        agg = lax.dot_general(oh_dst, msg, (((0,), (0,)), ((), ())),
                              preferred_element_type=jnp.float32,
                              precision=hp)
        h2 = jnp.maximum(jnp.dot(h + agg, w2_ref[...],
                                 preferred_element_type=jnp.float32,
                                 precision=hp), 0.0)
        h2s = jnp.dot(oh_src, h2, preferred_element_type=jnp.float32,
                      precision=hp)
        h2d = jnp.dot(oh_dst, h2, preferred_element_type=jnp.float32,
                      precision=hp)

        w3v = w3_ref[...]
        logit = (jnp.dot(h2s * h2d, w3v[:_HID, :],
                         preferred_element_type=jnp.float32, precision=hp)
                 + es_v * w3v[_HID, 0])
        preds = 1.0 / (1.0 + jnp.exp(-logit))
        loss = -(el_v * jnp.log(preds + eps)
                 + (1.0 - el_v) * jnp.log(1.0 - preds + eps))
        num_pos = jnp.sum(el_v * mask)
        num_neg = jnp.sum(mask) - num_pos
        mw = jnp.where(el_v > 0.0, 1.0 / (num_pos + eps),
                       1.0 / (num_neg + eps))
        out_ref[...] = jnp.sum(loss * mw * mask).reshape(1, 1)

    return pl.pallas_call(
        body,
        out_shape=jax.ShapeDtypeStruct((1, 1), jnp.float32),
        compiler_params=pltpu.CompilerParams(
            vmem_limit_bytes=63 * 1024 * 1024),
    )(contrib, src, dst, es, el, ev, W1, W2, W3)


def kernel(feature_maps, multiview_centers, edge_scores, edge_labels,
           edge_valid, edge_indices, W1, W2, W3):
    num_nodes = _V * _P

    fm4 = feature_maps.reshape(_B * _V, _C, _H, _W)
    pts = multiview_centers[..., :2].reshape(-1, 2)
    xs = jnp.pad(pts[:, 0], (0, _NPAD - _NODES))
    ys = jnp.pad(pts[:, 1], (0, _NPAD - _NODES))

    offs = (jnp.arange(_B, dtype=edge_indices.dtype) * num_nodes)[:, None, None]
    ei = (edge_indices + offs).reshape(-1, 2).astype(jnp.int32)
    src = ei[:, 0:1]
    dst = ei[:, 1:2]
    es = edge_scores.reshape(-1, 1)
    el = edge_labels.reshape(-1, 1)
    ev = edge_valid.reshape(-1, 1)

    rows, wvec = _sc_gather_rows(fm4, xs, ys)

    out = _tc_finish(rows, wvec, src, dst, es, el, ev, W1, W2, W3)
    return out[0, 0]
